# Initial kernel scaffold; baseline (speedup 1.0000x reference)
import jax
import jax.numpy as jnp
from jax.experimental import pallas as pl


def kernel(x, gate, W, b, noise):
    raise NotImplementedError("write your pallas kernel here")


# speedup vs baseline: 2.3886x; 2.3886x over previous
import jax
import jax.numpy as jnp
from jax.experimental import pallas as pl


def _moe_block_kernel(x_ref, noise_ref, gate_ref, w_ref, b_ref, out_ref):
    xt = x_ref[...]
    logits = jnp.dot(xt, gate_ref[...], preferred_element_type=jnp.float32)
    g = jax.nn.softmax(logits, axis=-1)
    xp = (xt + noise_ref[...]).astype(jnp.bfloat16)
    acc = jnp.dot(g, b_ref[...], preferred_element_type=jnp.float32)
    for e in range(w_ref.shape[0]):
        ye = jnp.dot(xp, w_ref[e], preferred_element_type=jnp.float32)
        acc = acc + g[:, e:e + 1] * ye
    out_ref[...] = acc


def kernel(x, gate, W, b, noise):
    B, N, D = x.shape
    E = gate.shape[1]
    O = W.shape[2]
    T = B * N
    TT = 512
    xf = x.reshape(T, D)
    Wb = W.astype(jnp.bfloat16)
    nb = N // TT
    out = pl.pallas_call(
        _moe_block_kernel,
        grid=(T // TT,),
        in_specs=[
            pl.BlockSpec((TT, D), lambda i: (i, 0)),
            pl.BlockSpec((TT, D), lambda i: (i % nb, 0)),
            pl.BlockSpec((D, E), lambda i: (0, 0)),
            pl.BlockSpec((E, D, O), lambda i: (0, 0, 0)),
            pl.BlockSpec((E, O), lambda i: (0, 0)),
        ],
        out_specs=pl.BlockSpec((TT, O), lambda i: (i, 0)),
        out_shape=jax.ShapeDtypeStruct((T, O), jnp.float32),
    )(xf, noise, gate, Wb, b)
    return out.reshape(B, N, O)


# speedup vs baseline: 2.4681x; 1.0333x over previous
import jax
import jax.numpy as jnp
from jax.experimental import pallas as pl


def _moe_block_kernel(x_ref, noise_ref, gate_ref, w_ref, b_ref, out_ref):
    xt = x_ref[...]
    logits = jnp.dot(xt, gate_ref[...], preferred_element_type=jnp.float32)
    g = jax.nn.softmax(logits, axis=-1)
    xp = (xt + noise_ref[...]).astype(jnp.bfloat16)
    acc = jnp.zeros(out_ref.shape, jnp.float32)
    for e in range(w_ref.shape[0]):
        ye = jnp.dot(xp, w_ref[e], preferred_element_type=jnp.float32)
        acc = acc + g[:, e:e + 1] * (ye + b_ref[e:e + 1, :])
    out_ref[...] = acc


def kernel(x, gate, W, b, noise):
    B, N, D = x.shape
    E = gate.shape[1]
    O = W.shape[2]
    T = B * N
    TT = 1024
    xf = x.reshape(T, D)
    Wb = W.astype(jnp.bfloat16)
    nb = N // TT
    out = pl.pallas_call(
        _moe_block_kernel,
        grid=(T // TT,),
        in_specs=[
            pl.BlockSpec((TT, D), lambda i: (i, 0)),
            pl.BlockSpec((TT, D), lambda i: (i % nb, 0)),
            pl.BlockSpec((D, E), lambda i: (0, 0)),
            pl.BlockSpec((E, D, O), lambda i: (0, 0, 0)),
            pl.BlockSpec((E, O), lambda i: (0, 0)),
        ],
        out_specs=pl.BlockSpec((TT, O), lambda i: (i, 0)),
        out_shape=jax.ShapeDtypeStruct((T, O), jnp.float32),
    )(xf, noise, gate, Wb, b)
    return out.reshape(B, N, O)
